# Initial kernel scaffold; baseline (speedup 1.0000x reference)
import jax
import jax.numpy as jnp
from jax.experimental import pallas as pl


def kernel(loc_ids, loc_table, cluster_table, cluster_proj_w, freq_table, freq_proj_w, ln_gamma, ln_beta, loc_to_cluster, loc_freq_bucket):
    raise NotImplementedError("write your pallas kernel here")


# speedup vs baseline: 16.3619x; 16.3619x over previous
import functools

import jax
import jax.numpy as jnp
from jax import lax
from jax.experimental import pallas as pl
from jax.experimental.pallas import tpu as pltpu
from jax.experimental.pallas import tpu_sc as plsc

_LN_EPS = 1e-5


def _fuse_body(ct_ref, cw_ref, ft_ref, fw_ref, g_ref, b_ref,
               loc_ref, cid_ref, fid_ref, out_ref):
    r = loc_ref.shape[0]
    nc = ct_ref.shape[0]
    nf = ft_ref.shape[0]
    pc = jnp.dot(ct_ref[...], cw_ref[...],
                 preferred_element_type=jnp.float32) * 0.3
    pf = jnp.dot(ft_ref[...], fw_ref[...],
                 preferred_element_type=jnp.float32) * 0.2
    cid = cid_ref[0]
    fid = fid_ref[0]
    oht_c = (cid == lax.broadcasted_iota(jnp.int32, (nc, r), 0)
             ).astype(jnp.float32)
    oht_f = (fid == lax.broadcasted_iota(jnp.int32, (nf, r), 0)
             ).astype(jnp.float32)
    emb_c = lax.dot_general(oht_c, pc, (((0,), (0,)), ((), ())),
                            preferred_element_type=jnp.float32)
    emb_f = lax.dot_general(oht_f, pf, (((0,), (0,)), ((), ())),
                            preferred_element_type=jnp.float32)
    x = loc_ref[...] + emb_c + emb_f
    mean = jnp.mean(x, axis=-1, keepdims=True)
    xc = x - mean
    var = jnp.mean(xc * xc, axis=-1, keepdims=True)
    out_ref[...] = xc * lax.rsqrt(var + _LN_EPS) * g_ref[...] + b_ref[...]


def _build_fused_table(loc_table, ct, cw_t, ft, fw_t, gamma2, beta2,
                       cid3, fid3, block_rows):
    n, d = loc_table.shape
    nb = n // block_rows
    nc = ct.shape[0]
    nf = ft.shape[0]
    return pl.pallas_call(
        _fuse_body,
        grid=(nb,),
        in_specs=[
            pl.BlockSpec((nc, ct.shape[1]), lambda i: (0, 0)),
            pl.BlockSpec((cw_t.shape[0], d), lambda i: (0, 0)),
            pl.BlockSpec((nf, ft.shape[1]), lambda i: (0, 0)),
            pl.BlockSpec((fw_t.shape[0], d), lambda i: (0, 0)),
            pl.BlockSpec((1, d), lambda i: (0, 0)),
            pl.BlockSpec((1, d), lambda i: (0, 0)),
            pl.BlockSpec((block_rows, d), lambda i: (i, 0)),
            pl.BlockSpec((1, 1, block_rows), lambda i: (i, 0, 0)),
            pl.BlockSpec((1, 1, block_rows), lambda i: (i, 0, 0)),
        ],
        out_specs=pl.BlockSpec((block_rows, d), lambda i: (i, 0)),
        out_shape=jax.ShapeDtypeStruct((n, d), jnp.float32),
    )(ct, cw_t, ft, fw_t, gamma2, beta2, loc_table, cid3, fid3)


_NC, _NS, _LANES = 2, 16, 16
_NW = _NC * _NS
_CHUNK = 128


def _make_gather(n_tokens, d):
    per_w = n_tokens // _NW
    n_chunks = per_w // _CHUNK
    mesh = plsc.VectorSubcoreMesh(core_axis_name="c", subcore_axis_name="s")

    @functools.partial(
        pl.kernel,
        out_type=jax.ShapeDtypeStruct((n_tokens, d), jnp.float32),
        mesh=mesh,
        scratch_types=[
            pltpu.VMEM((n_chunks, _CHUNK), jnp.int32),
            pltpu.VMEM((_CHUNK, d), jnp.float32),
            pltpu.SemaphoreType.DMA,
        ],
    )
    def gather_k(table_hbm, idx_hbm, out_hbm, idx_v, rows_v, sem):
        wid = lax.axis_index("s") * _NC + lax.axis_index("c")
        pltpu.sync_copy(idx_hbm.at[wid], idx_v)
        base0 = wid * per_w

        def body(j, carry):
            pltpu.async_copy(table_hbm.at[idx_v.at[j]], rows_v, sem).wait()
            pltpu.sync_copy(rows_v,
                            out_hbm.at[pl.ds(base0 + j * _CHUNK, _CHUNK)])
            return carry

        lax.fori_loop(0, n_chunks, body, 0)

    return gather_k


def kernel(loc_ids, loc_table, cluster_table, cluster_proj_w, freq_table,
           freq_proj_w, ln_gamma, ln_beta, loc_to_cluster, loc_freq_bucket):
    b, s = loc_ids.shape
    n, d = loc_table.shape
    n_tokens = b * s

    nc = 64
    nf = 16
    ct = jnp.zeros((nc, cluster_table.shape[1]), jnp.float32
                   ).at[:cluster_table.shape[0]].set(cluster_table)
    ft = jnp.zeros((nf, freq_table.shape[1]), jnp.float32
                   ).at[:freq_table.shape[0]].set(freq_table)
    cw_t = cluster_proj_w.T
    fw_t = freq_proj_w.T
    gamma2 = ln_gamma.reshape(1, d)
    beta2 = ln_beta.reshape(1, d)

    block_rows = 2000
    nb = n // block_rows
    cid3 = loc_to_cluster.reshape(nb, 1, block_rows)
    fid3 = loc_freq_bucket.reshape(nb, 1, block_rows)

    fused = _build_fused_table(loc_table, ct, cw_t, ft, fw_t, gamma2, beta2,
                               cid3, fid3, block_rows)

    flat_ids = loc_ids.reshape(-1).astype(jnp.int32)
    idx3 = flat_ids.reshape(_NW, n_tokens // (_NW * _CHUNK), _CHUNK)
    out = _make_gather(n_tokens, d)(fused, idx3)
    return out.reshape(b, s, d)
